# baseline (speedup 1.0000x reference)
import functools
import math

import jax
import jax.numpy as jnp
from jax import lax
from jax.experimental import pallas as pl
from jax.experimental.pallas import tpu as pltpu
from jax.experimental.pallas import tpu_sc as plsc

H = 128
NC, NP, NS = 100000, 50000, 1000
N_T = [NC, NP, NS]
OFF = [0, NC, NC + NP]
NT_ALL = NC + NP + NS

SRC_T = [0, 1, 0, 2, 1, 2]
DST_T = [1, 0, 2, 0, 2, 1]
E_CNT = [600000, 600000, 400000, 400000, 100000, 100000]
DES = {0: (1, 3), 1: (0, 5), 2: (2, 4)}

NPAD_T = [100352, 50176, 1024]

SCRATCH_ROWS = 102528
DUMMY_ROW = 102400
ZROWS = 256
ZPS = SCRATCH_ROWS // 16

PASSES = [
    dict(members=[(1, 0, 100352), (2, 100352, 1024), (4, 101376, 1024)],
         ep=1114112),
    dict(members=[(3, 0, 100352)], ep=409600),
    dict(members=[(0, 0, 50176), (5, 50176, 50176)], ep=704512),
]
GS0 = [p["ep"] // 16384 for p in PASSES]
GS1 = [p["ep"] // 8192 for p in PASSES]
EOUT = [1, 2, 4, 3, 0, 5]

_MESH = plsc.VectorSubcoreMesh(core_axis_name="c", subcore_axis_name="s")
_SC_PARAMS = pltpu.CompilerParams(use_tc_tiling_on_sc=False)


def _zero_own(acc, zbuf, sid):
    zb = sid * ZPS

    def zk(k, carry):
        pltpu.sync_copy(zbuf, acc.at[pl.ds(zb + k * ZROWS, ZROWS)])
        return carry

    lax.fori_loop(0, 25, zk, 0)
    pltpu.sync_copy(zbuf.at[pl.ds(0, ZPS - 25 * ZROWS)],
                    acc.at[pl.ds(zb + 25 * ZROWS, ZPS - 25 * ZROWS)])


def _init_zbuf(zbuf):
    def zinit(i, carry):
        zbuf[i, :] = jnp.zeros((16,), jnp.float32)
        return carry
    lax.fori_loop(0, ZROWS, zinit, 0)


def _pipe_groups(tab, acc, G, nbuf, isrc, idst, rows, gsem, ssem, load_idx):
    def fire_gathers(b):
        for j in range(nbuf):
            pltpu.make_async_copy(tab.at[isrc.at[b, j]], rows.at[b, j],
                                  gsem.at[b]).start()

    def drain_gathers(b):
        for j in range(nbuf):
            pltpu.make_async_copy(tab.at[isrc.at[b, j]], rows.at[b, j],
                                  gsem.at[b]).wait()

    def fire_scatters(b):
        for j in range(nbuf):
            pltpu.make_async_copy(rows.at[b, j], acc.at[idst.at[b, j]],
                                  ssem.at[b]).start(add=True)

    def drain_scatters(b):
        for j in range(nbuf):
            pltpu.make_async_copy(rows.at[b, j], acc.at[idst.at[b, j]],
                                  ssem.at[b]).wait()

    load_idx(0, 0)
    fire_gathers(0)

    def body(g, carry):
        b = lax.rem(g, 2)
        drain_gathers(b)
        fire_scatters(b)

        @pl.when(g + 1 < G)
        def _next():
            bn = 1 - b

            @pl.when(g > 0)
            def _drain():
                drain_scatters(bn)

            load_idx(g + 1, bn)
            fire_gathers(bn)

        return carry

    lax.fori_loop(0, G, body, 0)
    drain_scatters((G - 2) % 2)
    drain_scatters((G - 1) % 2)


def _sc_l0_body(x16, s0, d0, s1, d1, s2, d2,
                o1, o2, o4, o3, o0, o5, acc, isrc, idst, rows, zbuf,
                gsem, ssem):
    c = lax.axis_index("c")
    sid = lax.axis_index("s")
    w = c * 16 + sid
    _init_zbuf(zbuf)
    srcs, dsts = [s0, s1, s2], [d0, d1, d2]
    outs = {1: o1, 2: o2, 4: o4, 3: o3, 0: o0, 5: o5}
    for p in range(3):
        _zero_own(acc, zbuf, sid)
        plsc.subcore_barrier()
        G = GS0[p]
        rpw = G * 4
        sp, dp = srcs[p], dsts[p]

        def load_idx(g, b):
            gb = w * rpw + g * 4
            pltpu.sync_copy(sp.at[pl.ds(gb, 4)], isrc.at[b])
            pltpu.sync_copy(dp.at[pl.ds(gb, 4)], idst.at[b])

        _pipe_groups(x16, acc, G, 4, isrc, idst, rows, gsem, ssem, load_idx)
        plsc.subcore_barrier()
        for (e, loff, npd) in PASSES[p]["members"]:
            sh = npd // 16
            pltpu.sync_copy(acc.at[pl.ds(loff + sid * sh, sh)],
                            outs[e].at[c, pl.ds(sid * sh, sh), :])
        plsc.subcore_barrier()


def _sc_l1_body(tab8, s0, d0, s1, d1, s2, d2,
                o1, o2, o4, o3, o0, o5, acc, isrc, idst, rows, zbuf,
                gsem, ssem):
    c = lax.axis_index("c")
    sid = lax.axis_index("s")
    _init_zbuf(zbuf)
    srcs, dsts = [s0, s1, s2], [d0, d1, d2]
    outs = {1: o1, 2: o2, 4: o4, 3: o3, 0: o0, 5: o5}
    for jh in range(4):
        hc = c * 4 + jh
        for p in range(3):
            _zero_own(acc, zbuf, sid)
            plsc.subcore_barrier()
            G = GS1[p]
            rps = G * 4
            sp, dp = srcs[p], dsts[p]

            def load_idx(g, b):
                gb = sid * rps + g * 4
                pltpu.sync_copy(sp.at[hc, pl.ds(gb, 4)], isrc.at[b])
                pltpu.sync_copy(dp.at[pl.ds(gb, 4)], idst.at[b])

            _pipe_groups(tab8, acc, G, 4, isrc, idst, rows, gsem, ssem,
                         load_idx)
            plsc.subcore_barrier()
            for (e, loff, npd) in PASSES[p]["members"]:
                sh = npd // 16
                pltpu.sync_copy(
                    acc.at[pl.ds(loff + sid * sh, sh)],
                    outs[e].at[pl.ds(sid * sh, sh), pl.ds(hc * 16, 16)])
            plsc.subcore_barrier()


_sc_l0 = pl.kernel(
    _sc_l0_body,
    out_type=tuple(jax.ShapeDtypeStruct((2, NPAD_T[DST_T[e]], 16), jnp.float32)
                   for e in EOUT),
    mesh=_MESH,
    scratch_types=[
        pltpu.VMEM_SHARED((SCRATCH_ROWS, 16), jnp.float32),
        pltpu.VMEM((2, 4, 128), jnp.int32),
        pltpu.VMEM((2, 4, 128), jnp.int32),
        pltpu.VMEM((2, 4, 128, 16), jnp.float32),
        pltpu.VMEM((ZROWS, 16), jnp.float32),
        pltpu.SemaphoreType.DMA((2,)),
        pltpu.SemaphoreType.DMA((2,)),
    ],
    compiler_params=_SC_PARAMS,
)

_sc_l1 = pl.kernel(
    _sc_l1_body,
    out_type=tuple(jax.ShapeDtypeStruct((NPAD_T[DST_T[e]], H), jnp.float32)
                   for e in EOUT),
    mesh=_MESH,
    scratch_types=[
        pltpu.VMEM_SHARED((SCRATCH_ROWS, 16), jnp.float32),
        pltpu.VMEM((2, 4, 128), jnp.int32),
        pltpu.VMEM((2, 4, 128), jnp.int32),
        pltpu.VMEM((2, 4, 128, 16), jnp.float32),
        pltpu.VMEM((ZROWS, 16), jnp.float32),
        pltpu.SemaphoreType.DMA((2,)),
        pltpu.SemaphoreType.DMA((2,)),
    ],
    compiler_params=_SC_PARAMS,
)


def _ln_relu(h, g, b):
    mu = jnp.mean(h, axis=-1, keepdims=True)
    var = jnp.mean((h - mu) ** 2, axis=-1, keepdims=True)
    h = (h - mu) * lax.rsqrt(var + 1e-5) * g + b
    return jnp.maximum(h, 0.0)


def _tc0_body(p0a, p1a, p0b, p1b, raw, A1, A2, Wself, bias, g, b, o):
    sa = p0a[...] + p1a[...]
    ma = sa / jnp.maximum(sa[:, 15:16], 1.0)
    sb = p0b[...] + p1b[...]
    mb = sb / jnp.maximum(sb[:, 15:16], 1.0)
    h = (jnp.dot(ma, A1[...], preferred_element_type=jnp.float32)
         + jnp.dot(mb, A2[...], preferred_element_type=jnp.float32)
         + jnp.dot(raw[...], Wself[...], preferred_element_type=jnp.float32)
         + bias[...])
    o[...] = _ln_relu(h, g[...], b[...])


def _tc1_body(sa, sb, q0a, q1a, q0b, q1b, h0, B1, B2, Wr1, bias, g, b, o):
    cnta = q0a[:, 15:16] + q1a[:, 15:16]
    cntb = q0b[:, 15:16] + q1b[:, 15:16]
    agga = sa[...] / jnp.maximum(cnta, 1.0)
    aggb = sb[...] / jnp.maximum(cntb, 1.0)
    h = (jnp.dot(agga, B1[...], preferred_element_type=jnp.float32)
         + jnp.dot(aggb, B2[...], preferred_element_type=jnp.float32)
         + jnp.dot(h0[...], Wr1[...], preferred_element_type=jnp.float32)
         + bias[...])
    o[...] = _ln_relu(h, g[...], b[...])


_BLK = 1024


def _rows_spec(width):
    return pl.BlockSpec((_BLK, width), lambda i: (i, 0))


def _full_spec(shape):
    return pl.BlockSpec(shape, lambda i: tuple(0 for _ in shape))


def _make_tc0(n_out):
    grid = (math.ceil(n_out / _BLK),)
    return pl.pallas_call(
        _tc0_body,
        grid=grid,
        in_specs=[_rows_spec(16)] * 5 + [
            _full_spec((16, H)), _full_spec((16, H)), _full_spec((16, H)),
            _full_spec((1, H)), _full_spec((1, H)), _full_spec((1, H))],
        out_specs=_rows_spec(H),
        out_shape=jax.ShapeDtypeStruct((n_out, H), jnp.float32),
    )


def _make_tc1(n_out):
    grid = (math.ceil(n_out / _BLK),)
    return pl.pallas_call(
        _tc1_body,
        grid=grid,
        in_specs=[_rows_spec(H), _rows_spec(H)] + [_rows_spec(16)] * 4 +
                 [_rows_spec(H),
                  _full_spec((H, H)), _full_spec((H, H)), _full_spec((H, H)),
                  _full_spec((1, H)), _full_spec((1, H)), _full_spec((1, H))],
        out_specs=_rows_spec(H),
        out_shape=jax.ShapeDtypeStruct((n_out, H), jnp.float32),
    )


_TC0 = [_make_tc0(n) for n in N_T]
_TC1 = [_make_tc1(n) for n in N_T]


def _pad16(x, npad):
    z = jnp.zeros((npad, 16), jnp.float32)
    z = z.at[:x.shape[0], :x.shape[1]].set(x)
    return z.at[:x.shape[0], 15].set(1.0)


def kernel(x_customer, x_product, x_store, Wc, bc, Wp, bp, Ws, bs, Wl, bl, Wr,
           ln_g, ln_b, edge_index_buys, edge_index_bought_by, edge_index_visits,
           edge_index_visited_by, edge_index_sold_at, edge_index_sells):
    edges = [edge_index_buys, edge_index_bought_by, edge_index_visits,
             edge_index_visited_by, edge_index_sold_at, edge_index_sells]
    raws = [x_customer, x_product, x_store]

    l0s, l1s, dsts = [], [], []
    for p in PASSES:
        sg = jnp.concatenate(
            [edges[e][0] + OFF[SRC_T[e]] for (e, _, _) in p["members"]])
        dl = jnp.concatenate(
            [edges[e][1] + loff for (e, loff, _) in p["members"]])
        padn = p["ep"] - sg.shape[0]
        sg = jnp.concatenate([sg, jnp.zeros((padn,), jnp.int32)])
        dl = jnp.concatenate([dl, jnp.full((padn,), DUMMY_ROW, jnp.int32)])
        l0s.append(sg.reshape(-1, 128))
        s8 = (sg * 8)[None, :] + jnp.arange(8, dtype=jnp.int32)[:, None]
        l1s.append(s8.reshape(8, -1, 128))
        dsts.append(dl.reshape(-1, 128))

    def wsrc_pad(t):
        W = [Wc, Wp, Ws][t]
        b = [bc, bp, bs][t]
        z = jnp.zeros((16, H), jnp.float32)
        z = z.at[:W.shape[0]].set(W)
        return z.at[15].set(b)

    WSP = [wsrc_pad(t) for t in range(3)]

    x16 = jnp.concatenate([_pad16(raws[t], N_T[t]) for t in range(3)], axis=0)
    raw16 = [_pad16(raws[t], NPAD_T[t]) for t in range(3)]

    l0o = _sc_l0(x16, l0s[0], dsts[0], l0s[1], dsts[1], l0s[2], dsts[2])
    q = {e: l0o[i] for i, e in enumerate(EOUT)}

    h0 = []
    for t in range(3):
        e1, e2 = DES[t]
        A1 = 0.5 * (WSP[SRC_T[e1]] @ Wl[0, e1])
        A2 = 0.5 * (WSP[SRC_T[e2]] @ Wl[0, e2])
        Wself = WSP[t] @ (0.5 * (Wr[0, e1] + Wr[0, e2]))
        bias = (0.5 * (bl[0, e1] + bl[0, e2])).reshape(1, H)
        h0.append(_TC0[t](
            q[e1][0], q[e1][1], q[e2][0], q[e2][1], raw16[t],
            A1, A2, Wself, bias,
            ln_g[0, t].reshape(1, H), ln_b[0, t].reshape(1, H)))

    tab8 = jnp.concatenate(h0, axis=0).reshape(NT_ALL * 8, 16)
    l1o = _sc_l1(tab8, l1s[0], dsts[0], l1s[1], dsts[1], l1s[2], dsts[2])
    sgm = {e: l1o[i] for i, e in enumerate(EOUT)}

    out = []
    for t in range(3):
        e1, e2 = DES[t]
        B1 = 0.5 * Wl[1, e1]
        B2 = 0.5 * Wl[1, e2]
        Wr1 = 0.5 * (Wr[1, e1] + Wr[1, e2])
        bias = (0.5 * (bl[1, e1] + bl[1, e2])).reshape(1, H)
        out.append(_TC1[t](
            sgm[e1], sgm[e2], q[e1][0], q[e1][1], q[e2][0], q[e2][1], h0[t],
            B1, B2, Wr1, bias,
            ln_g[1, t].reshape(1, H), ln_b[1, t].reshape(1, H)))
    return tuple(out)


# speedup vs baseline: 1.0343x; 1.0343x over previous
import functools
import math

import jax
import jax.numpy as jnp
from jax import lax
from jax.experimental import pallas as pl
from jax.experimental.pallas import tpu as pltpu
from jax.experimental.pallas import tpu_sc as plsc

H = 128
NC, NP, NS = 100000, 50000, 1000
N_T = [NC, NP, NS]
OFF = [0, NC, NC + NP]
NT_ALL = NC + NP + NS

SRC_T = [0, 1, 0, 2, 1, 2]
DST_T = [1, 0, 2, 0, 2, 1]
E_CNT = [600000, 600000, 400000, 400000, 100000, 100000]
DES = {0: (1, 3), 1: (0, 5), 2: (2, 4)}

NPAD_T = [100352, 50176, 1024]

SCRATCH_ROWS = 102528
DUMMY_ROW = 102400
ZROWS = 256
ZPS = SCRATCH_ROWS // 16

PASSES = [
    dict(members=[(1, 0, 100352), (2, 100352, 1024), (4, 101376, 1024)],
         ep=1114112),
    dict(members=[(3, 0, 100352)], ep=409600),
    dict(members=[(0, 0, 50176), (5, 50176, 50176)], ep=704512),
]
GS0 = [p["ep"] // 16384 for p in PASSES]
EOUT = [1, 2, 4, 3, 0, 5]

L1C = [dict(e=1, ep=606208), dict(e=3, ep=409600)]
L1P = [dict(e=0, ep=602112), dict(e=5, ep=102400)]
L1S_EP = 501760
P_ROWS, P_DUMMY = 50304, 50200
S_ROWS, S_DUMMY = 2112, 2048

_MESH = plsc.VectorSubcoreMesh(core_axis_name="c", subcore_axis_name="s")
_SC_PARAMS = pltpu.CompilerParams(use_tc_tiling_on_sc=False)


def _zero_own(acc, zbuf, sid):
    zb = sid * ZPS

    def zk(k, carry):
        pltpu.sync_copy(zbuf, acc.at[pl.ds(zb + k * ZROWS, ZROWS)])
        return carry

    lax.fori_loop(0, 25, zk, 0)
    pltpu.sync_copy(zbuf.at[pl.ds(0, ZPS - 25 * ZROWS)],
                    acc.at[pl.ds(zb + 25 * ZROWS, ZPS - 25 * ZROWS)])


def _init_zbuf(zbuf):
    def zinit(i, carry):
        zbuf[i, :] = jnp.zeros((16,), jnp.float32)
        return carry
    lax.fori_loop(0, ZROWS, zinit, 0)


def _pipe_groups(tab, acc, G, nbuf, isrc, idst, rows, gsem, ssem, load_idx,
                 do_scatter=True, do_gather=True):
    def fire_gathers(b):
        if not do_gather:
            return
        for j in range(nbuf):
            pltpu.make_async_copy(tab.at[isrc.at[b, j]], rows.at[b, j],
                                  gsem.at[b]).start()

    def drain_gathers(b):
        if not do_gather:
            return
        for j in range(nbuf):
            pltpu.make_async_copy(tab.at[isrc.at[b, j]], rows.at[b, j],
                                  gsem.at[b]).wait()

    def fire_scatters(b):
        if not do_scatter:
            return
        for j in range(nbuf):
            pltpu.make_async_copy(rows.at[b, j], acc.at[idst.at[b, j]],
                                  ssem.at[b]).start(add=True)

    def drain_scatters(b):
        if not do_scatter:
            return
        for j in range(nbuf):
            pltpu.make_async_copy(rows.at[b, j], acc.at[idst.at[b, j]],
                                  ssem.at[b]).wait()

    load_idx(0, 0)
    fire_gathers(0)

    def body(g, carry):
        b = lax.rem(g, 2)
        drain_gathers(b)
        fire_scatters(b)

        @pl.when(g + 1 < G)
        def _next():
            bn = 1 - b

            @pl.when(g > 0)
            def _drain():
                drain_scatters(bn)

            load_idx(g + 1, bn)
            fire_gathers(bn)

        return carry

    lax.fori_loop(0, G, body, 0)
    drain_scatters((G - 2) % 2)
    drain_scatters((G - 1) % 2)


def _sc_l0_body(x16, s0, d0, s1, d1, s2, d2,
                o1, o2, o4, o3, o0, o5, acc, isrc, idst, rows, zbuf,
                gsem, ssem):
    c = lax.axis_index("c")
    sid = lax.axis_index("s")
    w = c * 16 + sid
    _init_zbuf(zbuf)
    srcs, dsts = [s0, s1, s2], [d0, d1, d2]
    outs = {1: o1, 2: o2, 4: o4, 3: o3, 0: o0, 5: o5}
    for p in range(3):
        _zero_own(acc, zbuf, sid)
        plsc.subcore_barrier()
        G = GS0[p]
        rpw = G * 4
        sp, dp = srcs[p], dsts[p]

        def load_idx(g, b):
            gb = w * rpw + g * 4
            pltpu.sync_copy(sp.at[pl.ds(gb, 4)], isrc.at[b])
            pltpu.sync_copy(dp.at[pl.ds(gb, 4)], idst.at[b])

        _pipe_groups(x16, acc, G, 4, isrc, idst, rows, gsem, ssem, load_idx)
        plsc.subcore_barrier()
        for (e, loff, npd) in PASSES[p]["members"]:
            sh = npd // 16
            pltpu.sync_copy(acc.at[pl.ds(loff + sid * sh, sh)],
                            outs[e].at[c, pl.ds(sid * sh, sh), :])
        plsc.subcore_barrier()


def _sc_l1c_body(tab8, s1, d1, s3, d3, o1, o3, acc, isrc, idst, rows, zbuf,
                 gsem, ssem):
    c = lax.axis_index("c")
    sid = lax.axis_index("s")
    _init_zbuf(zbuf)
    passes = [(s1, d1, o1, L1C[0]["ep"]), (s3, d3, o3, L1C[1]["ep"])]
    for jh in range(4):
        hc = c * 4 + jh
        for (sp, dp, oref, ep) in passes:
            _zero_own(acc, zbuf, sid)
            plsc.subcore_barrier()
            G = ep // 8192
            rps = G * 4

            def load_idx(g, b):
                gb = sid * rps + g * 4
                pltpu.sync_copy(sp.at[hc, pl.ds(gb, 4)], isrc.at[b])
                pltpu.sync_copy(dp.at[pl.ds(gb, 4)], idst.at[b])

            _pipe_groups(tab8, acc, G, 4, isrc, idst, rows, gsem, ssem,
                         load_idx)
            plsc.subcore_barrier()
            sh = 100352 // 16
            pltpu.sync_copy(acc.at[pl.ds(sid * sh, sh)],
                            oref.at[pl.ds(sid * sh, sh), pl.ds(hc * 16, 16)])
            plsc.subcore_barrier()


def _sc_l1p_body(tab4, s0, d0, s5, d5, o0, o5, acc, isrc, idst, rows, zbuf,
                 gsem, ssem):
    c = lax.axis_index("c")
    sid = lax.axis_index("s")

    def zinit(i, carry):
        zbuf[i, 0, :] = jnp.zeros((16,), jnp.float32)
        zbuf[i, 1, :] = jnp.zeros((16,), jnp.float32)
        return carry

    lax.fori_loop(0, 128, zinit, 0)
    passes = [(s0, d0, o0, L1P[0]["ep"]), (s5, d5, o5, L1P[1]["ep"])]
    zps = P_ROWS // 16
    for jp in range(2):
        hp = c * 2 + jp
        for (sp, dp, oref, ep) in passes:
            zb = sid * zps

            def zk(k, carry):
                pltpu.sync_copy(zbuf, acc.at[pl.ds(zb + k * 128, 128)])
                return carry

            lax.fori_loop(0, 24, zk, 0)
            pltpu.sync_copy(zbuf.at[pl.ds(0, 72)],
                            acc.at[pl.ds(zb + 24 * 128, 72)])
            plsc.subcore_barrier()
            G = ep // 4096
            rps = G * 4

            def load_idx(g, b):
                gb = sid * rps + g * 4
                pltpu.sync_copy(sp.at[hp, pl.ds(gb, 4)], isrc.at[b])
                pltpu.sync_copy(dp.at[pl.ds(gb, 4)], idst.at[b])

            _pipe_groups(tab4, acc, G, 4, isrc, idst, rows, gsem, ssem,
                         load_idx)
            plsc.subcore_barrier()
            sh = 50176 // 16
            pltpu.sync_copy(acc.at[pl.ds(sid * sh, sh)],
                            oref.at[pl.ds(sid * sh, sh), hp])
            plsc.subcore_barrier()


def _sc_l1s_body(tab16, ss, ds_, o2, o4, acc, isrc, idst, rows, zbuf,
                 gsem, ssem):
    c = lax.axis_index("c")
    sid = lax.axis_index("s")
    w = c * 16 + sid

    def zinit(i, carry):
        for q in range(8):
            zbuf[i, q, :] = jnp.zeros((16,), jnp.float32)
        return carry

    lax.fori_loop(0, 33, zinit, 0)
    zb = sid * (S_ROWS // 16)
    for k in range(4):
        pltpu.sync_copy(zbuf, acc.at[pl.ds(zb + k * 33, 33)])
    plsc.subcore_barrier()
    G = L1S_EP // 2048
    rpw = G * 4

    def load_idx(g, b):
        gb = w * rpw + g * 4
        pltpu.sync_copy(ss.at[pl.ds(gb, 4)], isrc.at[b])
        pltpu.sync_copy(ds_.at[pl.ds(gb, 4)], idst.at[b])

    _pipe_groups(tab16, acc, G, 4, isrc, idst, rows, gsem, ssem, load_idx)
    plsc.subcore_barrier()
    pltpu.sync_copy(acc.at[pl.ds(sid * 64, 64)],
                    o2.at[c, pl.ds(sid * 64, 64)])
    pltpu.sync_copy(acc.at[pl.ds(1024 + sid * 64, 64)],
                    o4.at[c, pl.ds(sid * 64, 64)])


_sc_l0 = pl.kernel(
    _sc_l0_body,
    out_type=tuple(jax.ShapeDtypeStruct((2, NPAD_T[DST_T[e]], 16), jnp.float32)
                   for e in EOUT),
    mesh=_MESH,
    scratch_types=[
        pltpu.VMEM_SHARED((SCRATCH_ROWS, 16), jnp.float32),
        pltpu.VMEM((2, 4, 128), jnp.int32),
        pltpu.VMEM((2, 4, 128), jnp.int32),
        pltpu.VMEM((2, 4, 128, 16), jnp.float32),
        pltpu.VMEM((ZROWS, 16), jnp.float32),
        pltpu.SemaphoreType.DMA((2,)),
        pltpu.SemaphoreType.DMA((2,)),
    ],
    compiler_params=_SC_PARAMS,
)

_sc_l1c = pl.kernel(
    _sc_l1c_body,
    out_type=(jax.ShapeDtypeStruct((100352, H), jnp.float32),
              jax.ShapeDtypeStruct((100352, H), jnp.float32)),
    mesh=_MESH,
    scratch_types=[
        pltpu.VMEM_SHARED((SCRATCH_ROWS, 16), jnp.float32),
        pltpu.VMEM((2, 4, 128), jnp.int32),
        pltpu.VMEM((2, 4, 128), jnp.int32),
        pltpu.VMEM((2, 4, 128, 16), jnp.float32),
        pltpu.VMEM((ZROWS, 16), jnp.float32),
        pltpu.SemaphoreType.DMA((2,)),
        pltpu.SemaphoreType.DMA((2,)),
    ],
    compiler_params=_SC_PARAMS,
)

_sc_l1p = pl.kernel(
    _sc_l1p_body,
    out_type=(jax.ShapeDtypeStruct((50176, 4, 2, 16), jnp.float32),
              jax.ShapeDtypeStruct((50176, 4, 2, 16), jnp.float32)),
    mesh=_MESH,
    scratch_types=[
        pltpu.VMEM_SHARED((P_ROWS, 2, 16), jnp.float32),
        pltpu.VMEM((2, 4, 64), jnp.int32),
        pltpu.VMEM((2, 4, 64), jnp.int32),
        pltpu.VMEM((2, 4, 64, 2, 16), jnp.float32),
        pltpu.VMEM((128, 2, 16), jnp.float32),
        pltpu.SemaphoreType.DMA((2,)),
        pltpu.SemaphoreType.DMA((2,)),
    ],
    compiler_params=_SC_PARAMS,
)

_sc_l1s = pl.kernel(
    _sc_l1s_body,
    out_type=(jax.ShapeDtypeStruct((2, 1024, 8, 16), jnp.float32),
              jax.ShapeDtypeStruct((2, 1024, 8, 16), jnp.float32)),
    mesh=_MESH,
    scratch_types=[
        pltpu.VMEM_SHARED((S_ROWS, 8, 16), jnp.float32),
        pltpu.VMEM((2, 4, 16), jnp.int32),
        pltpu.VMEM((2, 4, 16), jnp.int32),
        pltpu.VMEM((2, 4, 16, 8, 16), jnp.float32),
        pltpu.VMEM((33, 8, 16), jnp.float32),
        pltpu.SemaphoreType.DMA((2,)),
        pltpu.SemaphoreType.DMA((2,)),
    ],
    compiler_params=_SC_PARAMS,
)


def _ln_relu(h, g, b):
    mu = jnp.mean(h, axis=-1, keepdims=True)
    var = jnp.mean((h - mu) ** 2, axis=-1, keepdims=True)
    h = (h - mu) * lax.rsqrt(var + 1e-5) * g + b
    return jnp.maximum(h, 0.0)


def _tc0_body(p0a, p1a, p0b, p1b, raw, A1, A2, Wself, bias, g, b, o):
    sa = p0a[...] + p1a[...]
    ma = sa / jnp.maximum(sa[:, 15:16], 1.0)
    sb = p0b[...] + p1b[...]
    mb = sb / jnp.maximum(sb[:, 15:16], 1.0)
    h = (jnp.dot(ma, A1[...], preferred_element_type=jnp.float32)
         + jnp.dot(mb, A2[...], preferred_element_type=jnp.float32)
         + jnp.dot(raw[...], Wself[...], preferred_element_type=jnp.float32)
         + bias[...])
    o[...] = _ln_relu(h, g[...], b[...])


def _tc1_body(sa, sb, q0a, q1a, q0b, q1b, h0, B1, B2, Wr1, bias, g, b, o):
    cnta = q0a[:, 15:16] + q1a[:, 15:16]
    cntb = q0b[:, 15:16] + q1b[:, 15:16]
    agga = sa[...] / jnp.maximum(cnta, 1.0)
    aggb = sb[...] / jnp.maximum(cntb, 1.0)
    h = (jnp.dot(agga, B1[...], preferred_element_type=jnp.float32)
         + jnp.dot(aggb, B2[...], preferred_element_type=jnp.float32)
         + jnp.dot(h0[...], Wr1[...], preferred_element_type=jnp.float32)
         + bias[...])
    o[...] = _ln_relu(h, g[...], b[...])


_BLK = 1024


def _rows_spec(width):
    return pl.BlockSpec((_BLK, width), lambda i: (i, 0))


def _full_spec(shape):
    return pl.BlockSpec(shape, lambda i: tuple(0 for _ in shape))


def _make_tc0(n_out):
    grid = (math.ceil(n_out / _BLK),)
    return pl.pallas_call(
        _tc0_body,
        grid=grid,
        in_specs=[_rows_spec(16)] * 5 + [
            _full_spec((16, H)), _full_spec((16, H)), _full_spec((16, H)),
            _full_spec((1, H)), _full_spec((1, H)), _full_spec((1, H))],
        out_specs=_rows_spec(H),
        out_shape=jax.ShapeDtypeStruct((n_out, H), jnp.float32),
    )


def _make_tc1(n_out):
    grid = (math.ceil(n_out / _BLK),)
    return pl.pallas_call(
        _tc1_body,
        grid=grid,
        in_specs=[_rows_spec(H), _rows_spec(H)] + [_rows_spec(16)] * 4 +
                 [_rows_spec(H),
                  _full_spec((H, H)), _full_spec((H, H)), _full_spec((H, H)),
                  _full_spec((1, H)), _full_spec((1, H)), _full_spec((1, H))],
        out_specs=_rows_spec(H),
        out_shape=jax.ShapeDtypeStruct((n_out, H), jnp.float32),
    )


_TC0 = [_make_tc0(n) for n in N_T]
_TC1 = [_make_tc1(n) for n in N_T]


def _pad16(x, npad):
    z = jnp.zeros((npad, 16), jnp.float32)
    z = z.at[:x.shape[0], :x.shape[1]].set(x)
    return z.at[:x.shape[0], 15].set(1.0)


def kernel(x_customer, x_product, x_store, Wc, bc, Wp, bp, Ws, bs, Wl, bl, Wr,
           ln_g, ln_b, edge_index_buys, edge_index_bought_by, edge_index_visits,
           edge_index_visited_by, edge_index_sold_at, edge_index_sells):
    edges = [edge_index_buys, edge_index_bought_by, edge_index_visits,
             edge_index_visited_by, edge_index_sold_at, edge_index_sells]
    raws = [x_customer, x_product, x_store]

    def _pad_pair(sg, dl, ep, dummy):
        padn = ep - sg.shape[0]
        sg = jnp.concatenate([sg, jnp.zeros((padn,), jnp.int32)])
        dl = jnp.concatenate([dl, jnp.full((padn,), dummy, jnp.int32)])
        return sg, dl

    l0s, dsts = [], []
    for p in PASSES:
        sg = jnp.concatenate(
            [edges[e][0] + OFF[SRC_T[e]] for (e, _, _) in p["members"]])
        dl = jnp.concatenate(
            [edges[e][1] + loff for (e, loff, _) in p["members"]])
        sg, dl = _pad_pair(sg, dl, p["ep"], DUMMY_ROW)
        l0s.append(sg.reshape(-1, 128))
        dsts.append(dl.reshape(-1, 128))

    def _mk(e, ep, scale, minor, dummy, srcoff=None, dstoff=0):
        sg = edges[e][0] + (OFF[SRC_T[e]] if srcoff is None else srcoff)
        dl = edges[e][1] + dstoff
        sg, dl = _pad_pair(sg, dl, ep, dummy)
        if scale > 1:
            s = ((sg * scale)[None, :]
                 + jnp.arange(scale, dtype=jnp.int32)[:, None])
            s = s.reshape(scale, -1, minor)
        else:
            s = sg.reshape(-1, minor)
        return s, dl.reshape(-1, minor)

    s1c, d1c = _mk(1, L1C[0]["ep"], 8, 128, DUMMY_ROW)
    s3c, d3c = _mk(3, L1C[1]["ep"], 8, 128, DUMMY_ROW)
    s0p, d0p = _mk(0, L1P[0]["ep"], 4, 64, P_DUMMY)
    s5p, d5p = _mk(5, L1P[1]["ep"], 4, 64, P_DUMMY)
    sgS = jnp.concatenate([edges[2][0], edges[4][0] + OFF[1]])
    dlS = jnp.concatenate([edges[2][1], edges[4][1] + 1024])
    sgS, dlS = _pad_pair(sgS, dlS, L1S_EP, S_DUMMY)
    sS, dS = sgS.reshape(-1, 16), dlS.reshape(-1, 16)

    def wsrc_pad(t):
        W = [Wc, Wp, Ws][t]
        b = [bc, bp, bs][t]
        z = jnp.zeros((16, H), jnp.float32)
        z = z.at[:W.shape[0]].set(W)
        return z.at[15].set(b)

    WSP = [wsrc_pad(t) for t in range(3)]

    x16 = jnp.concatenate([_pad16(raws[t], N_T[t]) for t in range(3)], axis=0)
    raw16 = [_pad16(raws[t], NPAD_T[t]) for t in range(3)]

    l0o = _sc_l0(x16, l0s[0], dsts[0], l0s[1], dsts[1], l0s[2], dsts[2])
    q = {e: l0o[i] for i, e in enumerate(EOUT)}

    h0 = []
    for t in range(3):
        e1, e2 = DES[t]
        A1 = 0.5 * (WSP[SRC_T[e1]] @ Wl[0, e1])
        A2 = 0.5 * (WSP[SRC_T[e2]] @ Wl[0, e2])
        Wself = WSP[t] @ (0.5 * (Wr[0, e1] + Wr[0, e2]))
        bias = (0.5 * (bl[0, e1] + bl[0, e2])).reshape(1, H)
        h0.append(_TC0[t](
            q[e1][0], q[e1][1], q[e2][0], q[e2][1], raw16[t],
            A1, A2, Wself, bias,
            ln_g[0, t].reshape(1, H), ln_b[0, t].reshape(1, H)))

    hall = jnp.concatenate(h0, axis=0)
    o1, o3 = _sc_l1c(hall.reshape(NT_ALL * 8, 16), s1c, d1c, s3c, d3c)
    o0, o5 = _sc_l1p(hall.reshape(NT_ALL * 4, 2, 16), s0p, d0p, s5p, d5p)
    o2, o4 = _sc_l1s(hall.reshape(NT_ALL, 8, 16), sS, dS)
    sgm = {
        1: o1, 3: o3,
        0: o0.reshape(50176, H), 5: o5.reshape(50176, H),
        2: (o2[0] + o2[1]).reshape(1024, H),
        4: (o4[0] + o4[1]).reshape(1024, H),
    }

    out = []
    for t in range(3):
        e1, e2 = DES[t]
        B1 = 0.5 * Wl[1, e1]
        B2 = 0.5 * Wl[1, e2]
        Wr1 = 0.5 * (Wr[1, e1] + Wr[1, e2])
        bias = (0.5 * (bl[1, e1] + bl[1, e2])).reshape(1, H)
        out.append(_TC1[t](
            sgm[e1], sgm[e2], q[e1][0], q[e1][1], q[e2][0], q[e2][1], h0[t],
            B1, B2, Wr1, bias,
            ln_g[1, t].reshape(1, H), ln_b[1, t].reshape(1, H)))
    return tuple(out)


# speedup vs baseline: 1.0441x; 1.0095x over previous
import functools
import math

import jax
import jax.numpy as jnp
from jax import lax
from jax.experimental import pallas as pl
from jax.experimental.pallas import tpu as pltpu
from jax.experimental.pallas import tpu_sc as plsc

H = 128
NC, NP, NS = 100000, 50000, 1000
N_T = [NC, NP, NS]
OFF = [0, NC, NC + NP]
NT_ALL = NC + NP + NS

SRC_T = [0, 1, 0, 2, 1, 2]
DST_T = [1, 0, 2, 0, 2, 1]
E_CNT = [600000, 600000, 400000, 400000, 100000, 100000]
DES = {0: (1, 3), 1: (0, 5), 2: (2, 4)}

NPAD_T = [100352, 50176, 1024]

SCRATCH_ROWS = 102528
DUMMY_ROW = 102400
ZROWS = 256
ZPS = SCRATCH_ROWS // 16

PASSES = [
    dict(members=[(1, 0, 100352), (2, 100352, 1024), (4, 101376, 1024)],
         ep=1114112),
    dict(members=[(3, 0, 100352)], ep=409600),
    dict(members=[(0, 0, 50176), (5, 50176, 50176)], ep=704512),
]
GS0 = [p["ep"] // 16384 for p in PASSES]
EOUT = [1, 2, 4, 3, 0, 5]

L1C = [dict(e=1, ep=606208), dict(e=3, ep=409600)]
L1P_EPS = [602112, 102400, 503808]
P_ROWS, P_DUMMY = 50304, 50200

_MESH = plsc.VectorSubcoreMesh(core_axis_name="c", subcore_axis_name="s")
_SC_PARAMS = pltpu.CompilerParams(use_tc_tiling_on_sc=False)


def _zero_own(acc, zbuf, sid, share, zrows):
    zb = sid * share
    n, tail = share // zrows, share % zrows

    def zk(k, carry):
        pltpu.sync_copy(zbuf, acc.at[pl.ds(zb + k * zrows, zrows)])
        return carry

    lax.fori_loop(0, n, zk, 0)
    if tail:
        pltpu.sync_copy(zbuf.at[pl.ds(0, tail)],
                        acc.at[pl.ds(zb + n * zrows, tail)])


def _init_zbuf(zbuf):
    def zinit(i, carry):
        zbuf[i, :] = jnp.zeros((16,), jnp.float32)
        return carry
    lax.fori_loop(0, ZROWS, zinit, 0)


def _pipe_groups(tab, acc, G, nbuf, isrc, idst, rows, gsem, ssem, load_idx,
                 do_scatter=True, do_gather=True):
    def fire_gathers(b):
        if not do_gather:
            return
        for j in range(nbuf):
            pltpu.make_async_copy(tab.at[isrc.at[b, j]], rows.at[b, j],
                                  gsem.at[b]).start()

    def drain_gathers(b):
        if not do_gather:
            return
        for j in range(nbuf):
            pltpu.make_async_copy(tab.at[isrc.at[b, j]], rows.at[b, j],
                                  gsem.at[b]).wait()

    def fire_scatters(b):
        if not do_scatter:
            return
        for j in range(nbuf):
            pltpu.make_async_copy(rows.at[b, j], acc.at[idst.at[b, j]],
                                  ssem.at[b]).start(add=True)

    def drain_scatters(b):
        if not do_scatter:
            return
        for j in range(nbuf):
            pltpu.make_async_copy(rows.at[b, j], acc.at[idst.at[b, j]],
                                  ssem.at[b]).wait()

    load_idx(0, 0)
    fire_gathers(0)

    def body(g, carry):
        b = lax.rem(g, 2)
        drain_gathers(b)
        fire_scatters(b)

        @pl.when(g + 1 < G)
        def _next():
            bn = 1 - b

            @pl.when(g > 0)
            def _drain():
                drain_scatters(bn)

            load_idx(g + 1, bn)
            fire_gathers(bn)

        return carry

    lax.fori_loop(0, G, body, 0)
    drain_scatters((G - 2) % 2)
    drain_scatters((G - 1) % 2)


def _sc_l0_body(x16, s0, d0, s1, d1, s2, d2,
                o1, o2, o4, o3, o0, o5, acc, isrc, idst, rows, zbuf,
                gsem, ssem):
    c = lax.axis_index("c")
    sid = lax.axis_index("s")
    w = c * 16 + sid
    _init_zbuf(zbuf)
    srcs, dsts = [s0, s1, s2], [d0, d1, d2]
    outs = {1: o1, 2: o2, 4: o4, 3: o3, 0: o0, 5: o5}
    for p in range(3):
        _zero_own(acc, zbuf, sid, 6400, ZROWS)
        plsc.subcore_barrier()
        G = GS0[p]
        rpw = G * 4
        sp, dp = srcs[p], dsts[p]

        def load_idx(g, b):
            gb = w * rpw + g * 4
            pltpu.sync_copy(sp.at[pl.ds(gb, 4)], isrc.at[b])
            pltpu.sync_copy(dp.at[pl.ds(gb, 4)], idst.at[b])

        _pipe_groups(x16, acc, G, 4, isrc, idst, rows, gsem, ssem, load_idx)
        plsc.subcore_barrier()
        for (e, loff, npd) in PASSES[p]["members"]:
            sh = npd // 16
            pltpu.sync_copy(acc.at[pl.ds(loff + sid * sh, sh)],
                            outs[e].at[c, pl.ds(sid * sh, sh), :])
        plsc.subcore_barrier()


def _sc_l1c_body(tab8, s1, d1, s3, d3, o1, o3, acc, isrc, idst, rows, zbuf,
                 gsem, ssem):
    c = lax.axis_index("c")
    sid = lax.axis_index("s")
    _init_zbuf(zbuf)
    passes = [(s1, d1, o1, L1C[0]["ep"]), (s3, d3, o3, L1C[1]["ep"])]
    for jh in range(4):
        hc = c * 4 + jh
        for (sp, dp, oref, ep) in passes:
            _zero_own(acc, zbuf, sid, 6272, ZROWS)
            plsc.subcore_barrier()
            G = ep // 8192
            rps = G * 4

            def load_idx(g, b):
                gb = sid * rps + g * 4
                pltpu.sync_copy(sp.at[hc, pl.ds(gb, 4)], isrc.at[b])
                pltpu.sync_copy(dp.at[pl.ds(gb, 4)], idst.at[b])

            _pipe_groups(tab8, acc, G, 4, isrc, idst, rows, gsem, ssem,
                         load_idx)
            plsc.subcore_barrier()
            sh = 100352 // 16
            pltpu.sync_copy(acc.at[pl.ds(sid * sh, sh)],
                            oref.at[pl.ds(sid * sh, sh), pl.ds(hc * 16, 16)])
            plsc.subcore_barrier()


def _sc_l1p_body(tab4, s0, d0, s5, d5, ss, ds_, o0, o5, o2, o4,
                 acc, isrc, idst, rows, zbuf, gsem, ssem):
    c = lax.axis_index("c")
    sid = lax.axis_index("s")

    def zinit(i, carry):
        zbuf[i, 0, :] = jnp.zeros((16,), jnp.float32)
        zbuf[i, 1, :] = jnp.zeros((16,), jnp.float32)
        return carry

    lax.fori_loop(0, 128, zinit, 0)
    passes = [
        (s0, d0, L1P_EPS[0], 3136, [(o0, 0, 50176)]),
        (s5, d5, L1P_EPS[1], 3136, [(o5, 0, 50176)]),
        (ss, ds_, L1P_EPS[2], 132, [(o2, 0, 1024), (o4, 1024, 1024)]),
    ]
    for jp in range(2):
        hp = c * 2 + jp
        for (sp, dp, ep, zshare, members) in passes:
            _zero_own(acc, zbuf, sid, zshare, 128)
            plsc.subcore_barrier()
            G = ep // 4096
            rps = G * 4

            def load_idx(g, b):
                gb = sid * rps + g * 4
                pltpu.sync_copy(sp.at[hp, pl.ds(gb, 4)], isrc.at[b])
                pltpu.sync_copy(dp.at[pl.ds(gb, 4)], idst.at[b])

            _pipe_groups(tab4, acc, G, 4, isrc, idst, rows, gsem, ssem,
                         load_idx)
            plsc.subcore_barrier()
            for (oref, loff, nrows) in members:
                sh = nrows // 16
                pltpu.sync_copy(acc.at[pl.ds(loff + sid * sh, sh)],
                                oref.at[pl.ds(sid * sh, sh), hp])
            plsc.subcore_barrier()


_sc_l0 = pl.kernel(
    _sc_l0_body,
    out_type=tuple(jax.ShapeDtypeStruct((2, NPAD_T[DST_T[e]], 16), jnp.float32)
                   for e in EOUT),
    mesh=_MESH,
    scratch_types=[
        pltpu.VMEM_SHARED((SCRATCH_ROWS, 16), jnp.float32),
        pltpu.VMEM((2, 4, 128), jnp.int32),
        pltpu.VMEM((2, 4, 128), jnp.int32),
        pltpu.VMEM((2, 4, 128, 16), jnp.float32),
        pltpu.VMEM((ZROWS, 16), jnp.float32),
        pltpu.SemaphoreType.DMA((2,)),
        pltpu.SemaphoreType.DMA((2,)),
    ],
    compiler_params=_SC_PARAMS,
)

_sc_l1c = pl.kernel(
    _sc_l1c_body,
    out_type=(jax.ShapeDtypeStruct((100352, H), jnp.float32),
              jax.ShapeDtypeStruct((100352, H), jnp.float32)),
    mesh=_MESH,
    scratch_types=[
        pltpu.VMEM_SHARED((SCRATCH_ROWS, 16), jnp.float32),
        pltpu.VMEM((2, 4, 128), jnp.int32),
        pltpu.VMEM((2, 4, 128), jnp.int32),
        pltpu.VMEM((2, 4, 128, 16), jnp.float32),
        pltpu.VMEM((ZROWS, 16), jnp.float32),
        pltpu.SemaphoreType.DMA((2,)),
        pltpu.SemaphoreType.DMA((2,)),
    ],
    compiler_params=_SC_PARAMS,
)

_sc_l1p = pl.kernel(
    _sc_l1p_body,
    out_type=(jax.ShapeDtypeStruct((50176, 4, 2, 16), jnp.float32),
              jax.ShapeDtypeStruct((50176, 4, 2, 16), jnp.float32),
              jax.ShapeDtypeStruct((1024, 4, 2, 16), jnp.float32),
              jax.ShapeDtypeStruct((1024, 4, 2, 16), jnp.float32)),
    mesh=_MESH,
    scratch_types=[
        pltpu.VMEM_SHARED((P_ROWS, 2, 16), jnp.float32),
        pltpu.VMEM((2, 4, 64), jnp.int32),
        pltpu.VMEM((2, 4, 64), jnp.int32),
        pltpu.VMEM((2, 4, 64, 2, 16), jnp.float32),
        pltpu.VMEM((128, 2, 16), jnp.float32),
        pltpu.SemaphoreType.DMA((2,)),
        pltpu.SemaphoreType.DMA((2,)),
    ],
    compiler_params=_SC_PARAMS,
)


def _ln_relu(h, g, b):
    mu = jnp.mean(h, axis=-1, keepdims=True)
    var = jnp.mean((h - mu) ** 2, axis=-1, keepdims=True)
    h = (h - mu) * lax.rsqrt(var + 1e-5) * g + b
    return jnp.maximum(h, 0.0)


def _tc0_body(p0a, p1a, p0b, p1b, raw, A1, A2, Wself, bias, g, b, o):
    sa = p0a[...] + p1a[...]
    ma = sa / jnp.maximum(sa[:, 15:16], 1.0)
    sb = p0b[...] + p1b[...]
    mb = sb / jnp.maximum(sb[:, 15:16], 1.0)
    h = (jnp.dot(ma, A1[...], preferred_element_type=jnp.float32)
         + jnp.dot(mb, A2[...], preferred_element_type=jnp.float32)
         + jnp.dot(raw[...], Wself[...], preferred_element_type=jnp.float32)
         + bias[...])
    o[...] = _ln_relu(h, g[...], b[...])


def _tc1_body(sa, sb, q0a, q1a, q0b, q1b, h0, B1, B2, Wr1, bias, g, b, o):
    cnta = q0a[:, 15:16] + q1a[:, 15:16]
    cntb = q0b[:, 15:16] + q1b[:, 15:16]
    agga = sa[...] / jnp.maximum(cnta, 1.0)
    aggb = sb[...] / jnp.maximum(cntb, 1.0)
    h = (jnp.dot(agga, B1[...], preferred_element_type=jnp.float32)
         + jnp.dot(aggb, B2[...], preferred_element_type=jnp.float32)
         + jnp.dot(h0[...], Wr1[...], preferred_element_type=jnp.float32)
         + bias[...])
    o[...] = _ln_relu(h, g[...], b[...])


_BLK = 1024


def _rows_spec(width):
    return pl.BlockSpec((_BLK, width), lambda i: (i, 0))


def _full_spec(shape):
    return pl.BlockSpec(shape, lambda i: tuple(0 for _ in shape))


def _make_tc0(n_out):
    grid = (math.ceil(n_out / _BLK),)
    return pl.pallas_call(
        _tc0_body,
        grid=grid,
        in_specs=[_rows_spec(16)] * 5 + [
            _full_spec((16, H)), _full_spec((16, H)), _full_spec((16, H)),
            _full_spec((1, H)), _full_spec((1, H)), _full_spec((1, H))],
        out_specs=_rows_spec(H),
        out_shape=jax.ShapeDtypeStruct((n_out, H), jnp.float32),
    )


def _make_tc1(n_out):
    grid = (math.ceil(n_out / _BLK),)
    return pl.pallas_call(
        _tc1_body,
        grid=grid,
        in_specs=[_rows_spec(H), _rows_spec(H)] + [_rows_spec(16)] * 4 +
                 [_rows_spec(H),
                  _full_spec((H, H)), _full_spec((H, H)), _full_spec((H, H)),
                  _full_spec((1, H)), _full_spec((1, H)), _full_spec((1, H))],
        out_specs=_rows_spec(H),
        out_shape=jax.ShapeDtypeStruct((n_out, H), jnp.float32),
    )


_TC0 = [_make_tc0(n) for n in N_T]
_TC1 = [_make_tc1(n) for n in N_T]


def _pad16(x, npad):
    z = jnp.zeros((npad, 16), jnp.float32)
    z = z.at[:x.shape[0], :x.shape[1]].set(x)
    return z.at[:x.shape[0], 15].set(1.0)


def kernel(x_customer, x_product, x_store, Wc, bc, Wp, bp, Ws, bs, Wl, bl, Wr,
           ln_g, ln_b, edge_index_buys, edge_index_bought_by, edge_index_visits,
           edge_index_visited_by, edge_index_sold_at, edge_index_sells):
    edges = [edge_index_buys, edge_index_bought_by, edge_index_visits,
             edge_index_visited_by, edge_index_sold_at, edge_index_sells]
    raws = [x_customer, x_product, x_store]

    def _pad_pair(sg, dl, ep, dummy):
        padn = ep - sg.shape[0]
        sg = jnp.concatenate([sg, jnp.zeros((padn,), jnp.int32)])
        dl = jnp.concatenate([dl, jnp.full((padn,), dummy, jnp.int32)])
        return sg, dl

    l0s, dsts = [], []
    for p in PASSES:
        sg = jnp.concatenate(
            [edges[e][0] + OFF[SRC_T[e]] for (e, _, _) in p["members"]])
        dl = jnp.concatenate(
            [edges[e][1] + loff for (e, loff, _) in p["members"]])
        sg, dl = _pad_pair(sg, dl, p["ep"], DUMMY_ROW)
        l0s.append(sg.reshape(-1, 128))
        dsts.append(dl.reshape(-1, 128))

    def _mk(e, ep, scale, minor, dummy, srcoff=None, dstoff=0):
        sg = edges[e][0] + (OFF[SRC_T[e]] if srcoff is None else srcoff)
        dl = edges[e][1] + dstoff
        sg, dl = _pad_pair(sg, dl, ep, dummy)
        if scale > 1:
            s = ((sg * scale)[None, :]
                 + jnp.arange(scale, dtype=jnp.int32)[:, None])
            s = s.reshape(scale, -1, minor)
        else:
            s = sg.reshape(-1, minor)
        return s, dl.reshape(-1, minor)

    s1c, d1c = _mk(1, L1C[0]["ep"], 8, 128, DUMMY_ROW)
    s3c, d3c = _mk(3, L1C[1]["ep"], 8, 128, DUMMY_ROW)
    s0p, d0p = _mk(0, L1P_EPS[0], 4, 64, P_DUMMY)
    s5p, d5p = _mk(5, L1P_EPS[1], 4, 64, P_DUMMY)
    sgS = jnp.concatenate([edges[2][0], edges[4][0] + OFF[1]])
    dlS = jnp.concatenate([edges[2][1], edges[4][1] + 1024])
    sgS, dlS = _pad_pair(sgS, dlS, L1P_EPS[2], P_DUMMY)
    sS = ((sgS * 4)[None, :]
          + jnp.arange(4, dtype=jnp.int32)[:, None]).reshape(4, -1, 64)
    dS = dlS.reshape(-1, 64)

    def wsrc_pad(t):
        W = [Wc, Wp, Ws][t]
        b = [bc, bp, bs][t]
        z = jnp.zeros((16, H), jnp.float32)
        z = z.at[:W.shape[0]].set(W)
        return z.at[15].set(b)

    WSP = [wsrc_pad(t) for t in range(3)]

    x16 = jnp.concatenate([_pad16(raws[t], N_T[t]) for t in range(3)], axis=0)
    raw16 = [_pad16(raws[t], NPAD_T[t]) for t in range(3)]

    l0o = _sc_l0(x16, l0s[0], dsts[0], l0s[1], dsts[1], l0s[2], dsts[2])
    q = {e: l0o[i] for i, e in enumerate(EOUT)}

    h0 = []
    for t in range(3):
        e1, e2 = DES[t]
        A1 = 0.5 * (WSP[SRC_T[e1]] @ Wl[0, e1])
        A2 = 0.5 * (WSP[SRC_T[e2]] @ Wl[0, e2])
        Wself = WSP[t] @ (0.5 * (Wr[0, e1] + Wr[0, e2]))
        bias = (0.5 * (bl[0, e1] + bl[0, e2])).reshape(1, H)
        h0.append(_TC0[t](
            q[e1][0], q[e1][1], q[e2][0], q[e2][1], raw16[t],
            A1, A2, Wself, bias,
            ln_g[0, t].reshape(1, H), ln_b[0, t].reshape(1, H)))

    hall = jnp.concatenate(h0, axis=0)
    o1, o3 = _sc_l1c(hall.reshape(NT_ALL * 8, 16), s1c, d1c, s3c, d3c)
    o0, o5, o2, o4 = _sc_l1p(hall.reshape(NT_ALL * 4, 2, 16),
                             s0p, d0p, s5p, d5p, sS, dS)
    sgm = {
        1: o1, 3: o3,
        0: o0.reshape(50176, H), 5: o5.reshape(50176, H),
        2: o2.reshape(1024, H), 4: o4.reshape(1024, H),
    }

    out = []
    for t in range(3):
        e1, e2 = DES[t]
        B1 = 0.5 * Wl[1, e1]
        B2 = 0.5 * Wl[1, e2]
        Wr1 = 0.5 * (Wr[1, e1] + Wr[1, e2])
        bias = (0.5 * (bl[1, e1] + bl[1, e2])).reshape(1, H)
        out.append(_TC1[t](
            sgm[e1], sgm[e2], q[e1][0], q[e1][1], q[e2][0], q[e2][1], h0[t],
            B1, B2, Wr1, bias,
            ln_g[1, t].reshape(1, H), ln_b[1, t].reshape(1, H)))
    return tuple(out)


# speedup vs baseline: 1.1241x; 1.0766x over previous
import functools
import math

import jax
import jax.numpy as jnp
from jax import lax
from jax.experimental import pallas as pl
from jax.experimental.pallas import tpu as pltpu
from jax.experimental.pallas import tpu_sc as plsc

H = 128
NC, NP, NS = 100000, 50000, 1000
N_T = [NC, NP, NS]
OFF = [0, NC, NC + NP]
NT_ALL = NC + NP + NS

SRC_T = [0, 1, 0, 2, 1, 2]
DST_T = [1, 0, 2, 0, 2, 1]
E_CNT = [600000, 600000, 400000, 400000, 100000, 100000]
DES = {0: (1, 3), 1: (0, 5), 2: (2, 4)}

NPAD_T = [100352, 50176, 1024]

SCRATCH_ROWS = 102528
DUMMY_ROW = 102400
ZROWS = 256
ZPS = SCRATCH_ROWS // 16

PASSES = [
    dict(members=[(1, 0, 100352), (2, 100352, 1024), (4, 101376, 1024)],
         ep=1114112),
    dict(members=[(3, 0, 100352)], ep=409600),
    dict(members=[(0, 0, 50176), (5, 50176, 50176)], ep=704512),
]
GS0 = [p["ep"] // 16384 for p in PASSES]
EOUT = [1, 2, 4, 3, 0, 5]

L1C = [dict(e=1, ep=606208), dict(e=3, ep=409600)]
C_ROWS, C_DUMMY = 100416, 100400
L1P_EPS = [602112, 102400, 503808]
P_ROWS, P_DUMMY = 50304, 50200

_MESH = plsc.VectorSubcoreMesh(core_axis_name="c", subcore_axis_name="s")
_SC_PARAMS = pltpu.CompilerParams(use_tc_tiling_on_sc=False)


def _zero_own(acc, zbuf, sid, share, zrows):
    zb = sid * share
    n, tail = share // zrows, share % zrows

    def zk(k, carry):
        pltpu.sync_copy(zbuf, acc.at[pl.ds(zb + k * zrows, zrows)])
        return carry

    lax.fori_loop(0, n, zk, 0)
    if tail:
        pltpu.sync_copy(zbuf.at[pl.ds(0, tail)],
                        acc.at[pl.ds(zb + n * zrows, tail)])


def _init_zbuf(zbuf, n):
    def zinit(i, carry):
        zbuf[i, :] = jnp.zeros((16,), jnp.float32)
        return carry
    lax.fori_loop(0, n, zinit, 0)


def _pipe_groups(tab, acc, G, nbuf, isrc, idst, rows, gsem, ssem, load_idx,
                 nslots=2):
    def fire_gathers(b):
        for j in range(nbuf):
            pltpu.make_async_copy(tab.at[isrc.at[b, j]], rows.at[b, j],
                                  gsem.at[b]).start()

    def drain_gathers(b):
        for j in range(nbuf):
            pltpu.make_async_copy(tab.at[isrc.at[b, j]], rows.at[b, j],
                                  gsem.at[b]).wait()

    def fire_scatters(b):
        for j in range(nbuf):
            pltpu.make_async_copy(rows.at[b, j], acc.at[idst.at[b, j]],
                                  ssem.at[b]).start(add=True)

    def drain_scatters(b):
        for j in range(nbuf):
            pltpu.make_async_copy(rows.at[b, j], acc.at[idst.at[b, j]],
                                  ssem.at[b]).wait()

    for g0 in range(min(nslots - 1, G)):
        load_idx(g0, g0)
        fire_gathers(g0)

    def body(g, carry):
        b = lax.rem(g, nslots)
        drain_gathers(b)
        fire_scatters(b)
        gn = g + nslots - 1

        @pl.when(gn < G)
        def _next():
            bn = lax.rem(gn, nslots)

            @pl.when(g > 0)
            def _drain():
                drain_scatters(bn)

            load_idx(gn, bn)
            fire_gathers(bn)

        return carry

    lax.fori_loop(0, G, body, 0)
    for q in range(min(nslots, G)):
        drain_scatters((G - 1 - q) % nslots)


def _sc_l0_body(x16, s0, d0, s1, d1, s2, d2,
                o1, o2, o4, o3, o0, o5, acc, isrc, idst, rows, zbuf,
                gsem, ssem):
    c = lax.axis_index("c")
    sid = lax.axis_index("s")
    w = c * 16 + sid
    _init_zbuf(zbuf, ZROWS)
    srcs, dsts = [s0, s1, s2], [d0, d1, d2]
    outs = {1: o1, 2: o2, 4: o4, 3: o3, 0: o0, 5: o5}
    for p in range(3):
        _zero_own(acc, zbuf, sid, 6400, ZROWS)
        plsc.subcore_barrier()
        G = GS0[p]
        rpw = G * 4
        sp, dp = srcs[p], dsts[p]

        def load_idx(g, b):
            gb = w * rpw + g * 4
            pltpu.sync_copy(sp.at[pl.ds(gb, 4)], isrc.at[b])
            pltpu.sync_copy(dp.at[pl.ds(gb, 4)], idst.at[b])

        _pipe_groups(x16, acc, G, 4, isrc, idst, rows, gsem, ssem, load_idx)
        plsc.subcore_barrier()
        for (e, loff, npd) in PASSES[p]["members"]:
            sh = npd // 16
            pltpu.sync_copy(acc.at[pl.ds(loff + sid * sh, sh)],
                            outs[e].at[c, pl.ds(sid * sh, sh), :])
        plsc.subcore_barrier()


def _sc_l1c_body(tab8, s1, d1, s3, d3, o1, o3, acc, isrc, idst, rows, zbuf,
                 gsem, ssem):
    c = lax.axis_index("c")
    sid = lax.axis_index("s")
    _init_zbuf(zbuf, 128)
    passes = [(s1, d1, o1, L1C[0]["ep"]), (s3, d3, o3, L1C[1]["ep"])]
    for jh in range(4):
        hc = c * 4 + jh
        for (sp, dp, oref, ep) in passes:
            _zero_own(acc, zbuf, sid, 6272, 128)
            plsc.subcore_barrier()
            G = ep // 8192
            rps = G * 4

            def load_idx(g, b):
                gb = sid * rps + g * 4
                pltpu.sync_copy(sp.at[hc, pl.ds(gb, 4)], isrc.at[b])
                pltpu.sync_copy(dp.at[pl.ds(gb, 4)], idst.at[b])

            _pipe_groups(tab8, acc, G, 4, isrc, idst, rows, gsem, ssem,
                         load_idx, nslots=3)
            plsc.subcore_barrier()
            sh = 100352 // 16
            pltpu.sync_copy(acc.at[pl.ds(sid * sh, sh)],
                            oref.at[pl.ds(sid * sh, sh), pl.ds(hc * 16, 16)])
            plsc.subcore_barrier()


def _sc_l1p_body(tab4, s0, d0, s5, d5, ss, ds_, o0, o5, o2, o4,
                 acc, isrc, idst, rows, zbuf, gsem, ssem):
    c = lax.axis_index("c")
    sid = lax.axis_index("s")

    def zinit(i, carry):
        zbuf[i, 0, :] = jnp.zeros((16,), jnp.float32)
        zbuf[i, 1, :] = jnp.zeros((16,), jnp.float32)
        return carry

    lax.fori_loop(0, 128, zinit, 0)
    passes = [
        (s0, d0, L1P_EPS[0], 3136, [(o0, 0, 50176)]),
        (s5, d5, L1P_EPS[1], 3136, [(o5, 0, 50176)]),
        (ss, ds_, L1P_EPS[2], 132, [(o2, 0, 1024), (o4, 1024, 1024)]),
    ]
    for jp in range(2):
        hp = c * 2 + jp
        for (sp, dp, ep, zshare, members) in passes:
            _zero_own(acc, zbuf, sid, zshare, 128)
            plsc.subcore_barrier()
            G = ep // 4096
            rps = G * 4

            def load_idx(g, b):
                gb = sid * rps + g * 4
                pltpu.sync_copy(sp.at[hp, pl.ds(gb, 4)], isrc.at[b])
                pltpu.sync_copy(dp.at[pl.ds(gb, 4)], idst.at[b])

            _pipe_groups(tab4, acc, G, 4, isrc, idst, rows, gsem, ssem,
                         load_idx)
            plsc.subcore_barrier()
            for (oref, loff, nrows) in members:
                sh = nrows // 16
                pltpu.sync_copy(acc.at[pl.ds(loff + sid * sh, sh)],
                                oref.at[pl.ds(sid * sh, sh), hp])
            plsc.subcore_barrier()


_sc_l0 = pl.kernel(
    _sc_l0_body,
    out_type=tuple(jax.ShapeDtypeStruct((2, NPAD_T[DST_T[e]], 16), jnp.float32)
                   for e in EOUT),
    mesh=_MESH,
    scratch_types=[
        pltpu.VMEM_SHARED((SCRATCH_ROWS, 16), jnp.float32),
        pltpu.VMEM((2, 4, 128), jnp.int32),
        pltpu.VMEM((2, 4, 128), jnp.int32),
        pltpu.VMEM((2, 4, 128, 16), jnp.float32),
        pltpu.VMEM((ZROWS, 16), jnp.float32),
        pltpu.SemaphoreType.DMA((2,)),
        pltpu.SemaphoreType.DMA((2,)),
    ],
    compiler_params=_SC_PARAMS,
)

_sc_l1c = pl.kernel(
    _sc_l1c_body,
    out_type=(jax.ShapeDtypeStruct((100352, H), jnp.float32),
              jax.ShapeDtypeStruct((100352, H), jnp.float32)),
    mesh=_MESH,
    scratch_types=[
        pltpu.VMEM_SHARED((C_ROWS, 16), jnp.float32),
        pltpu.VMEM((3, 4, 128), jnp.int32),
        pltpu.VMEM((3, 4, 128), jnp.int32),
        pltpu.VMEM((3, 4, 128, 16), jnp.float32),
        pltpu.VMEM((128, 16), jnp.float32),
        pltpu.SemaphoreType.DMA((3,)),
        pltpu.SemaphoreType.DMA((3,)),
    ],
    compiler_params=_SC_PARAMS,
)

_sc_l1p = pl.kernel(
    _sc_l1p_body,
    out_type=(jax.ShapeDtypeStruct((50176, 4, 2, 16), jnp.float32),
              jax.ShapeDtypeStruct((50176, 4, 2, 16), jnp.float32),
              jax.ShapeDtypeStruct((1024, 4, 2, 16), jnp.float32),
              jax.ShapeDtypeStruct((1024, 4, 2, 16), jnp.float32)),
    mesh=_MESH,
    scratch_types=[
        pltpu.VMEM_SHARED((P_ROWS, 2, 16), jnp.float32),
        pltpu.VMEM((2, 4, 64), jnp.int32),
        pltpu.VMEM((2, 4, 64), jnp.int32),
        pltpu.VMEM((2, 4, 64, 2, 16), jnp.float32),
        pltpu.VMEM((128, 2, 16), jnp.float32),
        pltpu.SemaphoreType.DMA((2,)),
        pltpu.SemaphoreType.DMA((2,)),
    ],
    compiler_params=_SC_PARAMS,
)


def _ln_relu(h, g, b):
    mu = jnp.mean(h, axis=-1, keepdims=True)
    var = jnp.mean((h - mu) ** 2, axis=-1, keepdims=True)
    h = (h - mu) * lax.rsqrt(var + 1e-5) * g + b
    return jnp.maximum(h, 0.0)


def _tc0_body(p0a, p1a, p0b, p1b, raw, A1, A2, Wself, bias, g, b, o):
    sa = p0a[...] + p1a[...]
    ma = sa / jnp.maximum(sa[:, 15:16], 1.0)
    sb = p0b[...] + p1b[...]
    mb = sb / jnp.maximum(sb[:, 15:16], 1.0)
    h = (jnp.dot(ma, A1[...], preferred_element_type=jnp.float32)
         + jnp.dot(mb, A2[...], preferred_element_type=jnp.float32)
         + jnp.dot(raw[...], Wself[...], preferred_element_type=jnp.float32)
         + bias[...])
    o[...] = _ln_relu(h, g[...], b[...])


def _tc1_body(sa, sb, q0a, q1a, q0b, q1b, h0, B1, B2, Wr1, bias, g, b, o):
    cnta = q0a[:, 15:16] + q1a[:, 15:16]
    cntb = q0b[:, 15:16] + q1b[:, 15:16]
    agga = sa[...] / jnp.maximum(cnta, 1.0)
    aggb = sb[...] / jnp.maximum(cntb, 1.0)
    h = (jnp.dot(agga, B1[...], preferred_element_type=jnp.float32)
         + jnp.dot(aggb, B2[...], preferred_element_type=jnp.float32)
         + jnp.dot(h0[...], Wr1[...], preferred_element_type=jnp.float32)
         + bias[...])
    o[...] = _ln_relu(h, g[...], b[...])


_BLK = 1024


def _rows_spec(width):
    return pl.BlockSpec((_BLK, width), lambda i: (i, 0))


def _full_spec(shape):
    return pl.BlockSpec(shape, lambda i: tuple(0 for _ in shape))


def _make_tc0(n_out):
    grid = (math.ceil(n_out / _BLK),)
    return pl.pallas_call(
        _tc0_body,
        grid=grid,
        in_specs=[_rows_spec(16)] * 5 + [
            _full_spec((16, H)), _full_spec((16, H)), _full_spec((16, H)),
            _full_spec((1, H)), _full_spec((1, H)), _full_spec((1, H))],
        out_specs=_rows_spec(H),
        out_shape=jax.ShapeDtypeStruct((n_out, H), jnp.float32),
    )


def _make_tc1(n_out):
    grid = (math.ceil(n_out / _BLK),)
    return pl.pallas_call(
        _tc1_body,
        grid=grid,
        in_specs=[_rows_spec(H), _rows_spec(H)] + [_rows_spec(16)] * 4 +
                 [_rows_spec(H),
                  _full_spec((H, H)), _full_spec((H, H)), _full_spec((H, H)),
                  _full_spec((1, H)), _full_spec((1, H)), _full_spec((1, H))],
        out_specs=_rows_spec(H),
        out_shape=jax.ShapeDtypeStruct((n_out, H), jnp.float32),
    )


_TC0 = [_make_tc0(n) for n in N_T]
_TC1 = [_make_tc1(n) for n in N_T]


def _pad16(x, npad):
    z = jnp.zeros((npad, 16), jnp.float32)
    z = z.at[:x.shape[0], :x.shape[1]].set(x)
    return z.at[:x.shape[0], 15].set(1.0)


def kernel(x_customer, x_product, x_store, Wc, bc, Wp, bp, Ws, bs, Wl, bl, Wr,
           ln_g, ln_b, edge_index_buys, edge_index_bought_by, edge_index_visits,
           edge_index_visited_by, edge_index_sold_at, edge_index_sells):
    edges = [edge_index_buys, edge_index_bought_by, edge_index_visits,
             edge_index_visited_by, edge_index_sold_at, edge_index_sells]
    raws = [x_customer, x_product, x_store]

    def _pad_pair(sg, dl, ep, dummy):
        padn = ep - sg.shape[0]
        sg = jnp.concatenate([sg, jnp.zeros((padn,), jnp.int32)])
        dl = jnp.concatenate([dl, jnp.full((padn,), dummy, jnp.int32)])
        return sg, dl

    l0s, dsts = [], []
    for p in PASSES:
        sg = jnp.concatenate(
            [edges[e][0] + OFF[SRC_T[e]] for (e, _, _) in p["members"]])
        dl = jnp.concatenate(
            [edges[e][1] + loff for (e, loff, _) in p["members"]])
        sg, dl = _pad_pair(sg, dl, p["ep"], DUMMY_ROW)
        l0s.append(sg.reshape(-1, 128))
        dsts.append(dl.reshape(-1, 128))

    def _mk(e, ep, scale, minor, dummy, srcoff=None, dstoff=0):
        sg = edges[e][0] + (OFF[SRC_T[e]] if srcoff is None else srcoff)
        dl = edges[e][1] + dstoff
        sg, dl = _pad_pair(sg, dl, ep, dummy)
        if scale > 1:
            s = ((sg * scale)[None, :]
                 + jnp.arange(scale, dtype=jnp.int32)[:, None])
            s = s.reshape(scale, -1, minor)
        else:
            s = sg.reshape(-1, minor)
        return s, dl.reshape(-1, minor)

    s1c, d1c = _mk(1, L1C[0]["ep"], 8, 128, C_DUMMY)
    s3c, d3c = _mk(3, L1C[1]["ep"], 8, 128, C_DUMMY)
    s0p, d0p = _mk(0, L1P_EPS[0], 4, 64, P_DUMMY)
    s5p, d5p = _mk(5, L1P_EPS[1], 4, 64, P_DUMMY)
    sgS = jnp.concatenate([edges[2][0], edges[4][0] + OFF[1]])
    dlS = jnp.concatenate([edges[2][1], edges[4][1] + 1024])
    sgS, dlS = _pad_pair(sgS, dlS, L1P_EPS[2], P_DUMMY)
    sS = ((sgS * 4)[None, :]
          + jnp.arange(4, dtype=jnp.int32)[:, None]).reshape(4, -1, 64)
    dS = dlS.reshape(-1, 64)

    def wsrc_pad(t):
        W = [Wc, Wp, Ws][t]
        b = [bc, bp, bs][t]
        z = jnp.zeros((16, H), jnp.float32)
        z = z.at[:W.shape[0]].set(W)
        return z.at[15].set(b)

    WSP = [wsrc_pad(t) for t in range(3)]

    x16 = jnp.concatenate([_pad16(raws[t], N_T[t]) for t in range(3)], axis=0)
    raw16 = [_pad16(raws[t], NPAD_T[t]) for t in range(3)]

    l0o = _sc_l0(x16, l0s[0], dsts[0], l0s[1], dsts[1], l0s[2], dsts[2])
    q = {e: l0o[i] for i, e in enumerate(EOUT)}

    h0 = []
    for t in range(3):
        e1, e2 = DES[t]
        A1 = 0.5 * (WSP[SRC_T[e1]] @ Wl[0, e1])
        A2 = 0.5 * (WSP[SRC_T[e2]] @ Wl[0, e2])
        Wself = WSP[t] @ (0.5 * (Wr[0, e1] + Wr[0, e2]))
        bias = (0.5 * (bl[0, e1] + bl[0, e2])).reshape(1, H)
        h0.append(_TC0[t](
            q[e1][0], q[e1][1], q[e2][0], q[e2][1], raw16[t],
            A1, A2, Wself, bias,
            ln_g[0, t].reshape(1, H), ln_b[0, t].reshape(1, H)))

    hall = jnp.concatenate(h0, axis=0)
    o1, o3 = _sc_l1c(hall.reshape(NT_ALL * 8, 16), s1c, d1c, s3c, d3c)
    o0, o5, o2, o4 = _sc_l1p(hall.reshape(NT_ALL * 4, 2, 16),
                             s0p, d0p, s5p, d5p, sS, dS)
    sgm = {
        1: o1, 3: o3,
        0: o0.reshape(50176, H), 5: o5.reshape(50176, H),
        2: o2.reshape(1024, H), 4: o4.reshape(1024, H),
    }

    out = []
    for t in range(3):
        e1, e2 = DES[t]
        B1 = 0.5 * Wl[1, e1]
        B2 = 0.5 * Wl[1, e2]
        Wr1 = 0.5 * (Wr[1, e1] + Wr[1, e2])
        bias = (0.5 * (bl[1, e1] + bl[1, e2])).reshape(1, H)
        out.append(_TC1[t](
            sgm[e1], sgm[e2], q[e1][0], q[e1][1], q[e2][0], q[e2][1], h0[t],
            B1, B2, Wr1, bias,
            ln_g[1, t].reshape(1, H), ln_b[1, t].reshape(1, H)))
    return tuple(out)
